# baseline (speedup 1.0000x reference)
import functools

import jax
import jax.numpy as jnp
from jax import lax
from jax.experimental import pallas as pl
from jax.experimental.pallas import tpu as pltpu
from jax.experimental.pallas import tpu_sc as plsc

_INFO = plsc.get_sparse_core_info()
_NC = _INFO.num_cores
_NS = _INFO.num_subcores
_NW = _NC * _NS


@jax.jit
def _sc_lookup(idx_t, tbl_t):
    l_len, b_len = idx_t.shape
    d_len, v_len = tbl_t.shape
    rounds = d_len // _NW
    mesh = plsc.VectorSubcoreMesh(core_axis_name="c", subcore_axis_name="s")

    @functools.partial(
        pl.kernel,
        out_type=jax.ShapeDtypeStruct((l_len, d_len, b_len), tbl_t.dtype),
        mesh=mesh,
        scratch_types=[
            pltpu.VMEM((1, v_len), tbl_t.dtype),
            pltpu.VMEM((1, b_len), tbl_t.dtype),
            pltpu.VMEM((1, b_len), tbl_t.dtype),
            pltpu.VMEM((1, b_len), tbl_t.dtype),
            pltpu.VMEM((1, b_len), tbl_t.dtype),
            pltpu.VMEM((1, b_len), tbl_t.dtype),
            pltpu.SemaphoreType.DMA,
            pltpu.SemaphoreType.DMA,
            pltpu.SemaphoreType.DMA,
            pltpu.SemaphoreType.DMA,
            pltpu.SemaphoreType.DMA,
            pltpu.SemaphoreType.DMA,
        ],
        compiler_params=pltpu.CompilerParams(
            use_tc_tiling_on_sc=True, needs_layout_passes=False
        ),
    )
    def body(idx_hbm, tbl_hbm, out_hbm, row_v, b0, b1, b2, b3, b4,
             isem, w0, w1, w2, w3, w4):
        wid = lax.axis_index("s") * _NC + lax.axis_index("c")
        zeros = jnp.zeros((16,), jnp.int32)
        bufs = (b0, b1, b2, b3, b4)
        wsems = (w0, w1, w2, w3, w4)
        depth = len(bufs)

        for r in range(rounds):
            dd = r * _NW + wid
            pltpu.sync_copy(tbl_hbm.at[pl.ds(dd, 1)], row_v)
            for m in range(3):
                pltpu.async_copy(idx_hbm.at[pl.ds(m, 1)], bufs[m], isem)

            def block(o, carry):
                for m in range(depth):
                    l = o * depth + m
                    pltpu.make_async_copy(
                        idx_hbm.at[pl.ds(l, 1)], bufs[m], isem
                    ).wait()
                    buf = bufs[m]

                    @plsc.parallel_loop(0, b_len, 16, unroll=16)
                    def gather16(k):
                        iv = plsc.bitcast(buf[0, pl.ds(k, 16)], jnp.int32)
                        buf[0, pl.ds(k, 16)] = plsc.load_gather(
                            row_v, [zeros, iv]
                        )

                    pltpu.async_copy(
                        buf, out_hbm.at[l, pl.ds(dd, 1)], wsems[m]
                    )

                    m2 = (m + 3) % depth

                    @pl.when(l >= 2)
                    def _drain():
                        pltpu.make_async_copy(
                            bufs[m2], out_hbm.at[l - 2, pl.ds(dd, 1)],
                            wsems[m2],
                        ).wait()

                    @pl.when(l + 3 < l_len)
                    def _prefetch():
                        pltpu.async_copy(
                            idx_hbm.at[pl.ds(l + 3, 1)], bufs[m2], isem
                        )
                return carry

            lax.fori_loop(0, l_len // depth, block, 0)

            for l in (l_len - 2, l_len - 1):
                pltpu.make_async_copy(
                    bufs[l % depth], out_hbm.at[l, pl.ds(dd, 1)],
                    wsems[l % depth],
                ).wait()

    return body(idx_t, tbl_t)


def kernel(indices, table):
    idx_t = lax.bitcast_convert_type(
        jnp.transpose(indices.astype(jnp.int32)), jnp.float32
    )
    tbl_t = jnp.transpose(table)
    out_t = _sc_lookup(idx_t, tbl_t)
    return jnp.transpose(out_t, (2, 0, 1))


# speedup vs baseline: 1.0355x; 1.0355x over previous
import functools

import jax
import jax.numpy as jnp
from jax import lax
from jax.experimental import pallas as pl
from jax.experimental.pallas import tpu as pltpu
from jax.experimental.pallas import tpu_sc as plsc

_INFO = plsc.get_sparse_core_info()
_NC = _INFO.num_cores
_NS = _INFO.num_subcores
_NW = _NC * _NS

_IDEPTH = 5
_RDEPTH = 2


@jax.jit
def _sc_lookup(idx_t, tbl_t):
    l_len, b_len = idx_t.shape
    d_len, v_len = tbl_t.shape
    rounds = d_len // _NW
    unroll = _IDEPTH * _RDEPTH
    mesh = plsc.VectorSubcoreMesh(core_axis_name="c", subcore_axis_name="s")

    @functools.partial(
        pl.kernel,
        out_type=jax.ShapeDtypeStruct((l_len, d_len, b_len), tbl_t.dtype),
        mesh=mesh,
        scratch_types=[
            pltpu.VMEM((1, v_len), tbl_t.dtype),
            [pltpu.VMEM((1, b_len), jnp.int32) for _ in range(_IDEPTH)],
            [pltpu.VMEM((1, b_len), tbl_t.dtype) for _ in range(_RDEPTH)],
            pltpu.SemaphoreType.DMA,
            [pltpu.SemaphoreType.DMA for _ in range(_RDEPTH)],
        ],
        compiler_params=pltpu.CompilerParams(
            use_tc_tiling_on_sc=True, needs_layout_passes=False
        ),
    )
    def body(idx_hbm, tbl_hbm, out_hbm, row_v, ibufs, rbufs, isem, wsems):
        wid = lax.axis_index("s") * _NC + lax.axis_index("c")
        zeros = jnp.zeros((16,), jnp.int32)

        for r in range(rounds):
            dd = r * _NW + wid
            pltpu.sync_copy(tbl_hbm.at[pl.ds(dd, 1)], row_v)
            for m in range(_IDEPTH - 1):
                pltpu.async_copy(idx_hbm.at[pl.ds(m, 1)], ibufs[m], isem)

            def block(o, carry):
                for u in range(unroll):
                    l = o * unroll + u
                    mi = u % _IDEPTH
                    mr = u % _RDEPTH
                    pltpu.make_async_copy(
                        idx_hbm.at[pl.ds(l, 1)], ibufs[mi], isem
                    ).wait()

                    @pl.when(l >= _RDEPTH)
                    def _drain():
                        pltpu.make_async_copy(
                            rbufs[mr],
                            out_hbm.at[l - _RDEPTH, pl.ds(dd, 1)],
                            wsems[mr],
                        ).wait()

                    ib = ibufs[mi]
                    rb = rbufs[mr]

                    @plsc.parallel_loop(0, b_len, 16, unroll=8)
                    def gather16(k):
                        iv = ib[0, pl.ds(k, 16)]
                        rb[0, pl.ds(k, 16)] = plsc.load_gather(
                            row_v, [zeros, iv]
                        )

                    pltpu.async_copy(
                        rb, out_hbm.at[l, pl.ds(dd, 1)], wsems[mr]
                    )

                    @pl.when(l + _IDEPTH - 1 < l_len)
                    def _prefetch():
                        pltpu.async_copy(
                            idx_hbm.at[pl.ds(l + _IDEPTH - 1, 1)],
                            ibufs[(u + _IDEPTH - 1) % _IDEPTH], isem,
                        )
                return carry

            lax.fori_loop(0, l_len // unroll, block, 0)

            for l in (l_len - 2, l_len - 1):
                pltpu.make_async_copy(
                    rbufs[l % _RDEPTH], out_hbm.at[l, pl.ds(dd, 1)],
                    wsems[l % _RDEPTH],
                ).wait()

    return body(idx_t, tbl_t)


def kernel(indices, table):
    idx_t = jnp.transpose(indices.astype(jnp.int32))
    tbl_t = jnp.transpose(table)
    out_t = _sc_lookup(idx_t, tbl_t)
    return jnp.transpose(out_t, (2, 0, 1))
